# initial kernel scaffold (unmeasured)
import jax
import jax.numpy as jnp
from jax import lax
from jax.experimental import pallas as pl
from jax.experimental.pallas import tpu as pltpu

HALF_D = 2048
HALF_F = 4096
CH = 512
NCH = HALF_F // CH


def kernel(x, dy):
    xi = lax.axis_index("x")
    zi = lax.axis_index("z")

    xb = x.astype(jnp.bfloat16)
    dyb = dy.astype(jnp.bfloat16)
    dy_half = lax.dynamic_slice_in_dim(dyb, xi * HALF_F, HALF_F, axis=1)
    p = lax.dot_general(
        xb,
        dy_half,
        dimension_numbers=(((0,), (0,)), ((), ())),
        preferred_element_type=jnp.float32,
    )
    pk = lax.dynamic_slice_in_dim(p, zi * HALF_D, HALF_D, axis=0)
    pg = lax.dynamic_slice_in_dim(p, (1 - zi) * HALF_D, HALF_D, axis=0)
    pg = pg.astype(jnp.bfloat16)

    def body(
        pk_ref,
        pg_ref,
        out_ref,
        vq,
        vs,
        vrbf,
        vw,
        ldsem,
        stsem,
        zsend,
        zrecv,
        xsend,
        xrecv,
    ):
        my_x = lax.axis_index("x")
        my_y = lax.axis_index("y")
        my_z = lax.axis_index("z")
        z_peer = (my_x, my_y, 1 - my_z)
        x_peer = (1 - my_x, my_y, my_z)

        bsem = pltpu.get_barrier_semaphore()
        pl.semaphore_signal(
            bsem, inc=1, device_id=z_peer, device_id_type=pl.DeviceIdType.MESH
        )
        pl.semaphore_signal(
            bsem, inc=1, device_id=x_peer, device_id_type=pl.DeviceIdType.MESH
        )
        pl.semaphore_wait(bsem, 2)

        zr = pltpu.make_async_remote_copy(
            src_ref=pg_ref,
            dst_ref=vq,
            send_sem=zsend,
            recv_sem=zrecv,
            device_id=z_peer,
            device_id_type=pl.DeviceIdType.MESH,
        )
        zr.start()
        zr.wait()

        for j in range(NCH):
            s = j % 2
            ld = pltpu.make_async_copy(
                pk_ref.at[:, pl.ds(j * CH, CH)], vw.at[s], ldsem.at[s]
            )
            ld.start()
            ld.wait()
            r = vw[s] + vq[:, j * CH : (j + 1) * CH].astype(jnp.float32)
            vw[s] = r
            vrbf[:, j * CH : (j + 1) * CH] = r.astype(jnp.bfloat16)
            st = pltpu.make_async_copy(
                vw.at[s],
                out_ref.at[:, pl.ds(my_x * HALF_F + j * CH, CH)],
                stsem.at[s],
            )
            st.start()
            st.wait()

        xr = pltpu.make_async_remote_copy(
            src_ref=vrbf,
            dst_ref=vs,
            send_sem=xsend,
            recv_sem=xrecv,
            device_id=x_peer,
            device_id_type=pl.DeviceIdType.MESH,
        )
        xr.start()
        xr.wait()

        for j in range(NCH):
            s = j % 2
            vw[s] = vs[:, j * CH : (j + 1) * CH].astype(jnp.float32)
            st = pltpu.make_async_copy(
                vw.at[s],
                out_ref.at[:, pl.ds((1 - my_x) * HALF_F + j * CH, CH)],
                stsem.at[s],
            )
            st.start()
            st.wait()

    out = pl.pallas_call(
        body,
        out_shape=jax.ShapeDtypeStruct((HALF_D, 2 * HALF_F), jnp.float32),
        in_specs=[
            pl.BlockSpec(memory_space=pltpu.ANY),
            pl.BlockSpec(memory_space=pltpu.ANY),
        ],
        out_specs=pl.BlockSpec(memory_space=pltpu.ANY),
        scratch_shapes=[
            pltpu.VMEM((HALF_D, HALF_F), jnp.bfloat16),
            pltpu.VMEM((HALF_D, HALF_F), jnp.bfloat16),
            pltpu.VMEM((HALF_D, HALF_F), jnp.bfloat16),
            pltpu.VMEM((2, HALF_D, CH), jnp.float32),
            pltpu.SemaphoreType.DMA((2,)),
            pltpu.SemaphoreType.DMA((2,)),
            pltpu.SemaphoreType.DMA,
            pltpu.SemaphoreType.DMA,
            pltpu.SemaphoreType.DMA,
            pltpu.SemaphoreType.DMA,
        ],
        compiler_params=pltpu.CompilerParams(collective_id=0),
    )(pk, pg)
    return out


# baseline (device time: 734027 ns/iter reference)
import jax
import jax.numpy as jnp
from jax import lax
from jax.experimental import pallas as pl
from jax.experimental.pallas import tpu as pltpu

HALF_D = 2048
HALF_F = 4096
CH = 512
NCH = HALF_F // CH


def kernel(x, dy):
    xi = lax.axis_index("x")
    zi = lax.axis_index("z")

    xb = x.astype(jnp.bfloat16)
    dyb = dy.astype(jnp.bfloat16)
    dy_half = lax.dynamic_slice_in_dim(dyb, xi * HALF_F, HALF_F, axis=1)
    p = lax.dot_general(
        xb,
        dy_half,
        dimension_numbers=(((0,), (0,)), ((), ())),
        preferred_element_type=jnp.float32,
    )
    pk = lax.dynamic_slice_in_dim(p, zi * HALF_D, HALF_D, axis=0)
    pg = lax.dynamic_slice_in_dim(p, (1 - zi) * HALF_D, HALF_D, axis=0)
    pg = pg.astype(jnp.bfloat16)

    def body(
        pk_ref,
        pg_ref,
        out_ref,
        vq,
        vs,
        vrbf,
        vw,
        ldsem,
        stsem,
        zsend,
        zrecv,
        xsend,
        xrecv,
    ):
        my_x = lax.axis_index("x")
        my_y = lax.axis_index("y")
        my_z = lax.axis_index("z")
        z_peer = (my_x, my_y, 1 - my_z)
        x_peer = (1 - my_x, my_y, my_z)

        bsem = pltpu.get_barrier_semaphore()
        pl.semaphore_signal(
            bsem, inc=1, device_id=z_peer, device_id_type=pl.DeviceIdType.MESH
        )
        pl.semaphore_signal(
            bsem, inc=1, device_id=x_peer, device_id_type=pl.DeviceIdType.MESH
        )
        pl.semaphore_wait(bsem, 2)

        zr = pltpu.make_async_remote_copy(
            src_ref=pg_ref,
            dst_ref=vq,
            send_sem=zsend,
            recv_sem=zrecv,
            device_id=z_peer,
            device_id_type=pl.DeviceIdType.MESH,
        )
        zr.start()
        zr.wait()

        for j in range(NCH):
            s = j % 2
            ld = pltpu.make_async_copy(
                pk_ref.at[:, pl.ds(j * CH, CH)], vw.at[s], ldsem.at[s]
            )
            ld.start()
            ld.wait()
            r = vw[s] + vq[:, j * CH : (j + 1) * CH].astype(jnp.float32)
            vw[s] = r
            vrbf[:, j * CH : (j + 1) * CH] = r.astype(jnp.bfloat16)
            st = pltpu.make_async_copy(
                vw.at[s],
                out_ref.at[:, pl.ds(my_x * HALF_F + j * CH, CH)],
                stsem.at[s],
            )
            st.start()
            st.wait()

        xr = pltpu.make_async_remote_copy(
            src_ref=vrbf,
            dst_ref=vs,
            send_sem=xsend,
            recv_sem=xrecv,
            device_id=x_peer,
            device_id_type=pl.DeviceIdType.MESH,
        )
        xr.start()
        xr.wait()

        for j in range(NCH):
            s = j % 2
            vw[s] = vs[:, j * CH : (j + 1) * CH].astype(jnp.float32)
            st = pltpu.make_async_copy(
                vw.at[s],
                out_ref.at[:, pl.ds((1 - my_x) * HALF_F + j * CH, CH)],
                stsem.at[s],
            )
            st.start()
            st.wait()

    out = pl.pallas_call(
        body,
        out_shape=jax.ShapeDtypeStruct((HALF_D, 2 * HALF_F), jnp.float32),
        in_specs=[
            pl.BlockSpec(memory_space=pl.ANY),
            pl.BlockSpec(memory_space=pl.ANY),
        ],
        out_specs=pl.BlockSpec(memory_space=pl.ANY),
        scratch_shapes=[
            pltpu.VMEM((HALF_D, HALF_F), jnp.bfloat16),
            pltpu.VMEM((HALF_D, HALF_F), jnp.bfloat16),
            pltpu.VMEM((HALF_D, HALF_F), jnp.bfloat16),
            pltpu.VMEM((2, HALF_D, CH), jnp.float32),
            pltpu.SemaphoreType.DMA((2,)),
            pltpu.SemaphoreType.DMA((2,)),
            pltpu.SemaphoreType.DMA,
            pltpu.SemaphoreType.DMA,
            pltpu.SemaphoreType.DMA,
            pltpu.SemaphoreType.DMA,
        ],
        compiler_params=pltpu.CompilerParams(
            collective_id=0, vmem_limit_bytes=63 * 1024 * 1024
        ),
    )(pk, pg)
    return out


# device time: 468135 ns/iter; 1.5680x vs baseline; 1.5680x over previous
import jax
import jax.numpy as jnp
from jax import lax
from jax.experimental import pallas as pl
from jax.experimental.pallas import tpu as pltpu

K = 4096
D = 4096
HALF_D = 2048
HALF_F = 4096
PCH = 512
NPCH = HALF_F // PCH
CH = 256
NCH = HALF_F // CH


def _prep(x, dy):

    def body(x_ref, dy_ref, xb_ref, dyh_ref, f32s, b16s, ldsem, stsem):
        my_x = lax.axis_index("x")
        lds = []
        sts = {}
        for t in range(2 * NPCH):
            s = t % 2
            if t < NPCH:
                src = x_ref.at[:, pl.ds(t * PCH, PCH)]
            else:
                src = dy_ref.at[:, pl.ds(my_x * HALF_F + (t - NPCH) * PCH, PCH)]
            lds.append(pltpu.make_async_copy(src, f32s.at[s], ldsem.at[s]))
        lds[0].start()
        for t in range(2 * NPCH):
            s = t % 2
            if t + 1 < 2 * NPCH:
                lds[t + 1].start()
            lds[t].wait()
            if t >= 2:
                sts[t - 2].wait()
            b16s[s] = f32s[s].astype(jnp.bfloat16)
            if t < NPCH:
                dst = xb_ref.at[:, pl.ds(t * PCH, PCH)]
            else:
                dst = dyh_ref.at[:, pl.ds((t - NPCH) * PCH, PCH)]
            sts[t] = pltpu.make_async_copy(b16s.at[s], dst, stsem.at[s])
            sts[t].start()
        sts[2 * NPCH - 2].wait()
        sts[2 * NPCH - 1].wait()

    return pl.pallas_call(
        body,
        out_shape=(
            jax.ShapeDtypeStruct((K, D), jnp.bfloat16),
            jax.ShapeDtypeStruct((K, HALF_F), jnp.bfloat16),
        ),
        in_specs=[
            pl.BlockSpec(memory_space=pl.ANY),
            pl.BlockSpec(memory_space=pl.ANY),
        ],
        out_specs=(
            pl.BlockSpec(memory_space=pl.ANY),
            pl.BlockSpec(memory_space=pl.ANY),
        ),
        scratch_shapes=[
            pltpu.VMEM((2, K, PCH), jnp.float32),
            pltpu.VMEM((2, K, PCH), jnp.bfloat16),
            pltpu.SemaphoreType.DMA((2,)),
            pltpu.SemaphoreType.DMA((2,)),
        ],
    )(x, dy)


def _comm(p):

    def body(
        p_ref,
        out_ref,
        ldp,
        ldk,
        rf32,
        sf32,
        zsnd,
        vq,
        xsnd,
        vs,
        ldsemp,
        ldsemk,
        zssem,
        zrsem,
        xssem,
        xrsem,
        strsem,
        stssem,
        zcred,
        xcred,
    ):
        my_x = lax.axis_index("x")
        my_y = lax.axis_index("y")
        my_z = lax.axis_index("z")
        z_peer = (my_x, my_y, 1 - my_z)
        x_peer = (1 - my_x, my_y, my_z)
        myrow = my_z * HALF_D
        peerrow = (1 - my_z) * HALF_D

        bsem = pltpu.get_barrier_semaphore()
        pl.semaphore_signal(
            bsem, inc=1, device_id=z_peer, device_id_type=pl.DeviceIdType.MESH
        )
        pl.semaphore_signal(
            bsem, inc=1, device_id=x_peer, device_id_type=pl.DeviceIdType.MESH
        )
        pl.semaphore_wait(bsem, 2)

        def ldp_cp(j):
            return pltpu.make_async_copy(
                p_ref.at[pl.ds(peerrow, HALF_D), pl.ds(j * CH, CH)],
                ldp.at[j % 2],
                ldsemp.at[j % 2],
            )

        def ldk_cp(j):
            return pltpu.make_async_copy(
                p_ref.at[pl.ds(myrow, HALF_D), pl.ds(j * CH, CH)],
                ldk.at[j % 2],
                ldsemk.at[j % 2],
            )

        zrd = {}
        xrd = {}
        strcp = {}
        stscp = {}
        ldp_cp(0).start()
        ldk_cp(0).start()
        for j in range(NCH + 2):
            if j < NCH:
                s = j % 2
                ldp_cp(j).wait()
                if j >= 2:
                    zrd[j - 2].wait_send()
                zsnd[s] = ldp[s].astype(jnp.bfloat16)
                zrd[j] = pltpu.make_async_remote_copy(
                    src_ref=zsnd.at[s],
                    dst_ref=vq.at[s],
                    send_sem=zssem.at[s],
                    recv_sem=zrsem.at[s],
                    device_id=z_peer,
                    device_id_type=pl.DeviceIdType.MESH,
                )
                if j >= 2:
                    pl.semaphore_wait(zcred, 1)
                zrd[j].start()

            if 1 <= j <= NCH:
                i = j - 1
                si = i % 2
                zrd[i].wait_recv()
                ldk_cp(i).wait()
                if i >= 2:
                    strcp[i - 2].wait()
                r = ldk[si] + vq[si].astype(jnp.float32)
                rf32[si] = r
                if i < NCH - 2:
                    pl.semaphore_signal(
                        zcred,
                        inc=1,
                        device_id=z_peer,
                        device_id_type=pl.DeviceIdType.MESH,
                    )
                strcp[i] = pltpu.make_async_copy(
                    rf32.at[si],
                    out_ref.at[:, pl.ds(my_x * HALF_F + i * CH, CH)],
                    strsem.at[si],
                )
                strcp[i].start()
                if i >= 2:
                    xrd[i - 2].wait_send()
                xsnd[si] = r.astype(jnp.bfloat16)
                xrd[i] = pltpu.make_async_remote_copy(
                    src_ref=xsnd.at[si],
                    dst_ref=vs.at[si],
                    send_sem=xssem.at[si],
                    recv_sem=xrsem.at[si],
                    device_id=x_peer,
                    device_id_type=pl.DeviceIdType.MESH,
                )
                if i >= 2:
                    pl.semaphore_wait(xcred, 1)
                xrd[i].start()

            if 2 <= j <= NCH + 1:
                i = j - 2
                si = i % 2
                xrd[i].wait_recv()
                if i >= 1:
                    stscp[i - 1].wait()
                sf32[...] = vs[si].astype(jnp.float32)
                if i < NCH - 2:
                    pl.semaphore_signal(
                        xcred,
                        inc=1,
                        device_id=x_peer,
                        device_id_type=pl.DeviceIdType.MESH,
                    )
                stscp[i] = pltpu.make_async_copy(
                    sf32,
                    out_ref.at[:, pl.ds((1 - my_x) * HALF_F + i * CH, CH)],
                    stssem,
                )
                stscp[i].start()

            if j + 1 < NCH:
                ldp_cp(j + 1).start()
                ldk_cp(j + 1).start()

        strcp[NCH - 2].wait()
        strcp[NCH - 1].wait()
        stscp[NCH - 1].wait()
        zrd[NCH - 2].wait_send()
        zrd[NCH - 1].wait_send()
        xrd[NCH - 2].wait_send()
        xrd[NCH - 1].wait_send()

    return pl.pallas_call(
        body,
        out_shape=jax.ShapeDtypeStruct((HALF_D, 2 * HALF_F), jnp.float32),
        in_specs=[pl.BlockSpec(memory_space=pl.ANY)],
        out_specs=pl.BlockSpec(memory_space=pl.ANY),
        scratch_shapes=[
            pltpu.VMEM((2, HALF_D, CH), jnp.float32),
            pltpu.VMEM((2, HALF_D, CH), jnp.float32),
            pltpu.VMEM((2, HALF_D, CH), jnp.float32),
            pltpu.VMEM((HALF_D, CH), jnp.float32),
            pltpu.VMEM((2, HALF_D, CH), jnp.bfloat16),
            pltpu.VMEM((2, HALF_D, CH), jnp.bfloat16),
            pltpu.VMEM((2, HALF_D, CH), jnp.bfloat16),
            pltpu.VMEM((2, HALF_D, CH), jnp.bfloat16),
            pltpu.SemaphoreType.DMA((2,)),
            pltpu.SemaphoreType.DMA((2,)),
            pltpu.SemaphoreType.DMA((2,)),
            pltpu.SemaphoreType.DMA((2,)),
            pltpu.SemaphoreType.DMA((2,)),
            pltpu.SemaphoreType.DMA((2,)),
            pltpu.SemaphoreType.DMA((2,)),
            pltpu.SemaphoreType.DMA,
            pltpu.SemaphoreType.REGULAR,
            pltpu.SemaphoreType.REGULAR,
        ],
        compiler_params=pltpu.CompilerParams(
            collective_id=0, vmem_limit_bytes=63 * 1024 * 1024
        ),
    )(p)


def kernel(x, dy):
    xb, dyh = _prep(x, dy)
    p = lax.dot_general(
        xb,
        dyh,
        dimension_numbers=(((0,), (0,)), ((), ())),
        preferred_element_type=jnp.float32,
    )
    return _comm(p)
